# Initial kernel scaffold; baseline (speedup 1.0000x reference)
import functools

import jax
import jax.numpy as jnp
from jax import lax
from jax.experimental import pallas as pl
from jax.experimental.pallas import tpu as pltpu
from jax.experimental.pallas import tpu_sc as plsc

N = 10000
E = 320000
F = 128
HID = 128
C = 64

NC = 2
NS = 16
CH = 128

N_PAD = 10240
E_PAD = 323584
ROWS_PER_TILE = N_PAD // NS
CHUNKS_B = E_PAD // (NS * CH)
CHUNKS_D = E_PAD // (NC * NS * CH)

_MESH = plsc.VectorSubcoreMesh(
    core_axis_name="c", subcore_axis_name="s", num_cores=NC, num_subcores=NS
)


@functools.partial(
    pl.kernel,
    mesh=_MESH,
    out_type=(
        jax.ShapeDtypeStruct((N_PAD, F), jnp.float32),
        jax.ShapeDtypeStruct((N_PAD, F), jnp.float32),
        jax.ShapeDtypeStruct((N_PAD, 16), jnp.float32),
    ),
    scratch_types=[
        pltpu.VMEM_SHARED((N_PAD, F), jnp.float32),
        pltpu.VMEM_SHARED((N_PAD, 16), jnp.float32),
        pltpu.VMEM((CHUNKS_B, CH), jnp.int32),
        pltpu.VMEM((CHUNKS_B, CH), jnp.int32),
        pltpu.VMEM((CH, F), jnp.float32),
        pltpu.VMEM((CH, 16), jnp.float32),
        pltpu.SemaphoreType.DMA,
    ],
)
def _sc_layer1(masked_hbm, mask_hbm, src2d_hbm, dst2d_hbm, zeros128_hbm,
               zeros16_hbm, ones16_hbm,
               agg_out, denom_out, deg_out,
               acc, deg, sidx, didx, rows, ones_v, sem):
    c = lax.axis_index("c")
    s = lax.axis_index("s")
    nb = s * ROWS_PER_TILE

    pltpu.sync_copy(zeros128_hbm.at[pl.ds(nb, ROWS_PER_TILE)],
                    acc.at[pl.ds(nb, ROWS_PER_TILE)])
    pltpu.sync_copy(zeros16_hbm.at[pl.ds(nb, ROWS_PER_TILE)],
                    deg.at[pl.ds(nb, ROWS_PER_TILE)])
    pltpu.sync_copy(ones16_hbm, ones_v)

    pltpu.sync_copy(src2d_hbm.at[pl.ds(s * CHUNKS_B, CHUNKS_B)], sidx)
    pltpu.sync_copy(dst2d_hbm.at[pl.ds(s * CHUNKS_B, CHUNKS_B)], didx)

    plsc.subcore_barrier()

    @pl.when(c == 0)
    def _():
        def body(i, carry):
            pltpu.async_copy(masked_hbm.at[sidx.at[i]], rows, sem).wait()
            pltpu.sync_copy(rows, acc.at[didx.at[i]], add=True)
            return carry
        lax.fori_loop(0, CHUNKS_B, body, 0)

    @pl.when(c == 1)
    def _():
        def body(i, carry):
            pltpu.async_copy(mask_hbm.at[sidx.at[i]], rows, sem).wait()
            pltpu.sync_copy(rows, acc.at[didx.at[i]], add=True)
            pltpu.sync_copy(ones_v, deg.at[didx.at[i]], add=True)
            return carry
        lax.fori_loop(0, CHUNKS_B, body, 0)

    plsc.subcore_barrier()

    @pl.when(c == 0)
    def _():
        pltpu.sync_copy(acc.at[pl.ds(nb, ROWS_PER_TILE)],
                        agg_out.at[pl.ds(nb, ROWS_PER_TILE)])

    @pl.when(c == 1)
    def _():
        pltpu.sync_copy(acc.at[pl.ds(nb, ROWS_PER_TILE)],
                        denom_out.at[pl.ds(nb, ROWS_PER_TILE)])
        pltpu.sync_copy(deg.at[pl.ds(nb, ROWS_PER_TILE)],
                        deg_out.at[pl.ds(nb, ROWS_PER_TILE)])


@functools.partial(
    pl.kernel,
    mesh=_MESH,
    out_type=(
        jax.ShapeDtypeStruct((N_PAD, C), jnp.float32),
        jax.ShapeDtypeStruct((N_PAD, C), jnp.float32),
    ),
    scratch_types=[
        pltpu.VMEM_SHARED((N_PAD, C), jnp.float32),
        pltpu.VMEM((CHUNKS_D, CH), jnp.int32),
        pltpu.VMEM((CHUNKS_D, CH), jnp.int32),
        pltpu.VMEM((CH, C), jnp.float32),
        pltpu.SemaphoreType.DMA,
    ],
)
def _sc_layer2(g_hbm, src2d_hbm, dst2d_hbm, zeros64_hbm,
               p0_out, p1_out,
               acc, sidx, didx, rows, sem):
    c = lax.axis_index("c")
    s = lax.axis_index("s")
    nb = s * ROWS_PER_TILE

    pltpu.sync_copy(zeros64_hbm.at[pl.ds(nb, ROWS_PER_TILE)],
                    acc.at[pl.ds(nb, ROWS_PER_TILE)])

    wrow = (c * NS + s) * CHUNKS_D
    pltpu.sync_copy(src2d_hbm.at[pl.ds(wrow, CHUNKS_D)], sidx)
    pltpu.sync_copy(dst2d_hbm.at[pl.ds(wrow, CHUNKS_D)], didx)

    plsc.subcore_barrier()

    def body(i, carry):
        pltpu.async_copy(g_hbm.at[sidx.at[i]], rows, sem).wait()
        pltpu.sync_copy(rows, acc.at[didx.at[i]], add=True)
        return carry
    lax.fori_loop(0, CHUNKS_D, body, 0)

    plsc.subcore_barrier()

    @pl.when(c == 0)
    def _():
        pltpu.sync_copy(acc.at[pl.ds(nb, ROWS_PER_TILE)],
                        p0_out.at[pl.ds(nb, ROWS_PER_TILE)])

    @pl.when(c == 1)
    def _():
        pltpu.sync_copy(acc.at[pl.ds(nb, ROWS_PER_TILE)],
                        p1_out.at[pl.ds(nb, ROWS_PER_TILE)])


def _tc_masked_body(x_ref, m_ref, o_ref):
    o_ref[...] = x_ref[...] * m_ref[...]


def _tc_dense_body(agg_ref, den_ref, deg_ref, w1_ref, b1_ref, w2_ref, g_ref):
    a = agg_ref[...] / jnp.maximum(den_ref[...], 1.0)
    h = jnp.dot(a, w1_ref[...], preferred_element_type=jnp.float32) + b1_ref[...]
    h = jnp.maximum(h, 0.0)
    hw = jnp.dot(h, w2_ref[...], preferred_element_type=jnp.float32)
    dinv = lax.rsqrt(deg_ref[...][:, 0:1] + 1.0)
    g_ref[...] = dinv * hw


def _tc_final_body(p0_ref, p1_ref, g_ref, deg_ref, b2_ref, o_ref):
    dinv = lax.rsqrt(deg_ref[...][:, 0:1] + 1.0)
    o = dinv * (p0_ref[...] + p1_ref[...] + g_ref[...]) + b2_ref[...]
    m = jnp.max(o, axis=1, keepdims=True)
    z = o - m
    lse = jnp.log(jnp.sum(jnp.exp(z), axis=1, keepdims=True))
    o_ref[...] = z - lse


def kernel(x, edge_index, mask, W1, b1, W2, b2):
    src, dst = edge_index[0], edge_index[1]

    pad_n = N_PAD - N
    xp = jnp.pad(x, ((0, pad_n), (0, 0)))
    maskp = jnp.pad(mask, ((0, pad_n), (0, 0)))
    pad_e = E_PAD - E
    srcp = jnp.concatenate([src, jnp.full((pad_e,), N, jnp.int32)]).reshape(-1, CH)
    dstp = jnp.concatenate([dst, jnp.full((pad_e,), N, jnp.int32)]).reshape(-1, CH)
    zeros128 = jnp.zeros((N_PAD, F), jnp.float32)
    zeros64 = jnp.zeros((N_PAD, C), jnp.float32)
    zeros16 = jnp.zeros((N_PAD, 16), jnp.float32)
    ones16 = jnp.ones((CH, 16), jnp.float32)
    b1r = b1.reshape(1, HID)
    b2r = b2.reshape(1, C)

    grid = N_PAD // 1024
    masked = pl.pallas_call(
        _tc_masked_body,
        grid=(grid,),
        in_specs=[
            pl.BlockSpec((1024, F), lambda i: (i, 0)),
            pl.BlockSpec((1024, F), lambda i: (i, 0)),
        ],
        out_specs=pl.BlockSpec((1024, F), lambda i: (i, 0)),
        out_shape=jax.ShapeDtypeStruct((N_PAD, F), jnp.float32),
    )(xp, maskp)

    agg, denom, deg16 = _sc_layer1(masked, maskp, srcp, dstp,
                                   zeros128, zeros16, ones16)

    g = pl.pallas_call(
        _tc_dense_body,
        grid=(grid,),
        in_specs=[
            pl.BlockSpec((1024, F), lambda i: (i, 0)),
            pl.BlockSpec((1024, F), lambda i: (i, 0)),
            pl.BlockSpec((1024, 16), lambda i: (i, 0)),
            pl.BlockSpec((F, HID), lambda i: (0, 0)),
            pl.BlockSpec((1, HID), lambda i: (0, 0)),
            pl.BlockSpec((HID, C), lambda i: (0, 0)),
        ],
        out_specs=pl.BlockSpec((1024, C), lambda i: (i, 0)),
        out_shape=jax.ShapeDtypeStruct((N_PAD, C), jnp.float32),
    )(agg, denom, deg16, W1, b1r, W2)

    p0, p1 = _sc_layer2(g, srcp, dstp, zeros64)

    out = pl.pallas_call(
        _tc_final_body,
        grid=(10,),
        in_specs=[
            pl.BlockSpec((1000, C), lambda i: (i, 0)),
            pl.BlockSpec((1000, C), lambda i: (i, 0)),
            pl.BlockSpec((1000, C), lambda i: (i, 0)),
            pl.BlockSpec((1000, 16), lambda i: (i, 0)),
            pl.BlockSpec((1, C), lambda i: (0, 0)),
        ],
        out_specs=pl.BlockSpec((1000, C), lambda i: (i, 0)),
        out_shape=jax.ShapeDtypeStruct((N, C), jnp.float32),
    )(p0, p1, g, deg16, b2r)

    return out


# speedup vs baseline: 6.0033x; 6.0033x over previous
import functools

import jax
import jax.numpy as jnp
from jax import lax
from jax.experimental import pallas as pl
from jax.experimental.pallas import tpu as pltpu
from jax.experimental.pallas import tpu_sc as plsc

N = 10000
E = 320000
F = 128
HID = 128
C = 64

NC = 2
NS = 16
CH = 128

N_PAD = 10240
E_PAD = 327680
ROWS_PER_TILE = N_PAD // NS
CHUNKS_B = E_PAD // (NS * CH)
CHUNKS_D = E_PAD // (NC * NS * CH)

_MESH = plsc.VectorSubcoreMesh(
    core_axis_name="c", subcore_axis_name="s", num_cores=NC, num_subcores=NS
)


@functools.partial(
    pl.kernel,
    mesh=_MESH,
    out_type=(
        jax.ShapeDtypeStruct((N_PAD, F), jnp.float32),
        jax.ShapeDtypeStruct((N_PAD, F), jnp.float32),
    ),
    scratch_types=[
        pltpu.VMEM_SHARED((N_PAD, F), jnp.float32),
        pltpu.VMEM((8, CH), jnp.int32),
        pltpu.VMEM((8, CH), jnp.int32),
        pltpu.VMEM((CH, F), jnp.float32),
        pltpu.SemaphoreType.DMA,
    ],
)
def _sc_layer1(masked_hbm, mask_hbm, src2d_hbm, dst2d_hbm, zeros128_hbm,
               agg_out, denom_out,
               acc, sidx, didx, rows, sem):
    c = lax.axis_index("c")
    s = lax.axis_index("s")
    nb = s * ROWS_PER_TILE

    pltpu.sync_copy(zeros128_hbm.at[pl.ds(nb, ROWS_PER_TILE)],
                    acc.at[pl.ds(nb, ROWS_PER_TILE)])

    plsc.subcore_barrier()

    @pl.when(c == 0)
    def _():
        def body(j, carry):
            base = s * CHUNKS_B + j * 8
            pltpu.sync_copy(src2d_hbm.at[pl.ds(base, 8)], sidx)
            pltpu.sync_copy(dst2d_hbm.at[pl.ds(base, 8)], didx)
            for k in range(8):
                pltpu.async_copy(masked_hbm.at[sidx.at[k]], rows, sem).wait()
                pltpu.sync_copy(rows, acc.at[didx.at[k]], add=True)
            return carry
        lax.fori_loop(0, CHUNKS_B // 8, body, 0)

    @pl.when(c == 1)
    def _():
        def body(j, carry):
            base = s * CHUNKS_B + j * 8
            pltpu.sync_copy(src2d_hbm.at[pl.ds(base, 8)], sidx)
            pltpu.sync_copy(dst2d_hbm.at[pl.ds(base, 8)], didx)
            for k in range(8):
                pltpu.async_copy(mask_hbm.at[sidx.at[k]], rows, sem).wait()
                pltpu.sync_copy(rows, acc.at[didx.at[k]], add=True)
            return carry
        lax.fori_loop(0, CHUNKS_B // 8, body, 0)

    plsc.subcore_barrier()

    @pl.when(c == 0)
    def _():
        pltpu.sync_copy(acc.at[pl.ds(nb, ROWS_PER_TILE)],
                        agg_out.at[pl.ds(nb, ROWS_PER_TILE)])

    @pl.when(c == 1)
    def _():
        pltpu.sync_copy(acc.at[pl.ds(nb, ROWS_PER_TILE)],
                        denom_out.at[pl.ds(nb, ROWS_PER_TILE)])


@functools.partial(
    pl.kernel,
    mesh=_MESH,
    out_type=(
        jax.ShapeDtypeStruct((N_PAD, F), jnp.float32),
        jax.ShapeDtypeStruct((N_PAD, F), jnp.float32),
    ),
    scratch_types=[
        pltpu.VMEM_SHARED((N_PAD, F), jnp.float32),
        pltpu.VMEM((8, CH), jnp.int32),
        pltpu.VMEM((CH, F), jnp.float32),
    ],
)
def _sc_deg(dst2d_hbm, zeros128_hbm, ones128_hbm,
            d0_out, d1_out,
            acc, didx, ones_v):
    c = lax.axis_index("c")
    s = lax.axis_index("s")
    nb = s * ROWS_PER_TILE

    pltpu.sync_copy(zeros128_hbm.at[pl.ds(nb, ROWS_PER_TILE)],
                    acc.at[pl.ds(nb, ROWS_PER_TILE)])
    pltpu.sync_copy(ones128_hbm, ones_v)

    plsc.subcore_barrier()

    wrow = (c * NS + s) * CHUNKS_D

    def body(j, carry):
        pltpu.sync_copy(dst2d_hbm.at[pl.ds(wrow + j * 8, 8)], didx)
        for k in range(8):
            pltpu.sync_copy(ones_v, acc.at[didx.at[k]], add=True)
        return carry
    lax.fori_loop(0, CHUNKS_D // 8, body, 0)

    plsc.subcore_barrier()

    @pl.when(c == 0)
    def _():
        pltpu.sync_copy(acc.at[pl.ds(nb, ROWS_PER_TILE)],
                        d0_out.at[pl.ds(nb, ROWS_PER_TILE)])

    @pl.when(c == 1)
    def _():
        pltpu.sync_copy(acc.at[pl.ds(nb, ROWS_PER_TILE)],
                        d1_out.at[pl.ds(nb, ROWS_PER_TILE)])


@functools.partial(
    pl.kernel,
    mesh=_MESH,
    out_type=(
        jax.ShapeDtypeStruct((N_PAD, F), jnp.float32),
        jax.ShapeDtypeStruct((N_PAD, F), jnp.float32),
    ),
    scratch_types=[
        pltpu.VMEM_SHARED((N_PAD, F), jnp.float32),
        pltpu.VMEM((8, CH), jnp.int32),
        pltpu.VMEM((8, CH), jnp.int32),
        pltpu.VMEM((CH, F), jnp.float32),
        pltpu.SemaphoreType.DMA,
    ],
)
def _sc_layer2(g_hbm, src2d_hbm, dst2d_hbm, zeros128_hbm,
               p0_out, p1_out,
               acc, sidx, didx, rows, sem):
    c = lax.axis_index("c")
    s = lax.axis_index("s")
    nb = s * ROWS_PER_TILE

    pltpu.sync_copy(zeros128_hbm.at[pl.ds(nb, ROWS_PER_TILE)],
                    acc.at[pl.ds(nb, ROWS_PER_TILE)])

    plsc.subcore_barrier()

    wrow = (c * NS + s) * CHUNKS_D

    def body(j, carry):
        base = wrow + j * 8
        pltpu.sync_copy(src2d_hbm.at[pl.ds(base, 8)], sidx)
        pltpu.sync_copy(dst2d_hbm.at[pl.ds(base, 8)], didx)
        for k in range(8):
            pltpu.async_copy(g_hbm.at[sidx.at[k]], rows, sem).wait()
            pltpu.sync_copy(rows, acc.at[didx.at[k]], add=True)
        return carry
    lax.fori_loop(0, CHUNKS_D // 8, body, 0)

    plsc.subcore_barrier()

    @pl.when(c == 0)
    def _():
        pltpu.sync_copy(acc.at[pl.ds(nb, ROWS_PER_TILE)],
                        p0_out.at[pl.ds(nb, ROWS_PER_TILE)])

    @pl.when(c == 1)
    def _():
        pltpu.sync_copy(acc.at[pl.ds(nb, ROWS_PER_TILE)],
                        p1_out.at[pl.ds(nb, ROWS_PER_TILE)])


def _tc_masked_body(x_ref, m_ref, o_ref):
    o_ref[...] = x_ref[...] * m_ref[...]


def _tc_dense_body(agg_ref, den_ref, d0_ref, d1_ref, w1_ref, b1_ref, w2_ref,
                   g_ref):
    a = agg_ref[...] / jnp.maximum(den_ref[...], 1.0)
    h = jnp.dot(a, w1_ref[...], preferred_element_type=jnp.float32) + b1_ref[...]
    h = jnp.maximum(h, 0.0)
    hw = jnp.dot(h, w2_ref[...], preferred_element_type=jnp.float32)
    deg = d0_ref[...][:, 0:1] + d1_ref[...][:, 0:1]
    dinv = lax.rsqrt(deg + 1.0)
    gv = dinv * hw
    g_ref[...] = jnp.concatenate([gv, jnp.zeros_like(gv)], axis=1)


def _tc_final_body(p0_ref, p1_ref, g_ref, d0_ref, d1_ref, b2_ref, o_ref):
    deg = d0_ref[...][:, 0:1] + d1_ref[...][:, 0:1]
    dinv = lax.rsqrt(deg + 1.0)
    tot = (p0_ref[...] + p1_ref[...] + g_ref[...])[:, :C]
    o = dinv * tot + b2_ref[...]
    m = jnp.max(o, axis=1, keepdims=True)
    z = o - m
    lse = jnp.log(jnp.sum(jnp.exp(z), axis=1, keepdims=True))
    o_ref[...] = z - lse


def kernel(x, edge_index, mask, W1, b1, W2, b2):
    src, dst = edge_index[0], edge_index[1]

    pad_n = N_PAD - N
    xp = jnp.pad(x, ((0, pad_n), (0, 0)))
    maskp = jnp.pad(mask, ((0, pad_n), (0, 0)))
    pad_e = E_PAD - E
    srcp = jnp.concatenate([src, jnp.full((pad_e,), N, jnp.int32)]).reshape(-1, CH)
    dstp = jnp.concatenate([dst, jnp.full((pad_e,), N, jnp.int32)]).reshape(-1, CH)
    zeros128 = jnp.zeros((N_PAD, F), jnp.float32)
    ones128 = jnp.ones((CH, F), jnp.float32)
    b1r = b1.reshape(1, HID)
    b2r = b2.reshape(1, C)

    grid = N_PAD // 1024
    masked = pl.pallas_call(
        _tc_masked_body,
        grid=(grid,),
        in_specs=[
            pl.BlockSpec((1024, F), lambda i: (i, 0)),
            pl.BlockSpec((1024, F), lambda i: (i, 0)),
        ],
        out_specs=pl.BlockSpec((1024, F), lambda i: (i, 0)),
        out_shape=jax.ShapeDtypeStruct((N_PAD, F), jnp.float32),
    )(xp, maskp)

    d0, d1 = _sc_deg(dstp, zeros128, ones128)
    agg, denom = _sc_layer1(masked, maskp, srcp, dstp, zeros128)

    g = pl.pallas_call(
        _tc_dense_body,
        grid=(grid,),
        in_specs=[
            pl.BlockSpec((1024, F), lambda i: (i, 0)),
            pl.BlockSpec((1024, F), lambda i: (i, 0)),
            pl.BlockSpec((1024, F), lambda i: (i, 0)),
            pl.BlockSpec((1024, F), lambda i: (i, 0)),
            pl.BlockSpec((F, HID), lambda i: (0, 0)),
            pl.BlockSpec((1, HID), lambda i: (0, 0)),
            pl.BlockSpec((HID, C), lambda i: (0, 0)),
        ],
        out_specs=pl.BlockSpec((1024, F), lambda i: (i, 0)),
        out_shape=jax.ShapeDtypeStruct((N_PAD, F), jnp.float32),
    )(agg, denom, d0, d1, W1, b1r, W2)

    p0, p1 = _sc_layer2(g, srcp, dstp, zeros128)

    out = pl.pallas_call(
        _tc_final_body,
        grid=(10,),
        in_specs=[
            pl.BlockSpec((1000, F), lambda i: (i, 0)),
            pl.BlockSpec((1000, F), lambda i: (i, 0)),
            pl.BlockSpec((1000, F), lambda i: (i, 0)),
            pl.BlockSpec((1000, F), lambda i: (i, 0)),
            pl.BlockSpec((1000, F), lambda i: (i, 0)),
            pl.BlockSpec((1, C), lambda i: (0, 0)),
        ],
        out_specs=pl.BlockSpec((1000, C), lambda i: (i, 0)),
        out_shape=jax.ShapeDtypeStruct((N, C), jnp.float32),
    )(p0, p1, g, d0, d1, b2r)

    return out


# speedup vs baseline: 16.0652x; 2.6761x over previous
import functools

import jax
import jax.numpy as jnp
from jax import lax
from jax.experimental import pallas as pl
from jax.experimental.pallas import tpu as pltpu
from jax.experimental.pallas import tpu_sc as plsc

N = 10000
E = 320000
F = 128
HID = 128
C = 64

NC = 2
NS = 16
CH = 128

N_PAD = 10240
E_PAD = 327680
ROWS_PER_TILE = N_PAD // NS
CHUNKS_B = E_PAD // (NS * CH)
CHUNKS_D = E_PAD // (NC * NS * CH)

_MESH = plsc.VectorSubcoreMesh(
    core_axis_name="c", subcore_axis_name="s", num_cores=NC, num_subcores=NS
)


@functools.partial(
    pl.kernel,
    mesh=_MESH,
    out_type=(
        jax.ShapeDtypeStruct((N_PAD, F), jnp.float32),
        jax.ShapeDtypeStruct((N_PAD, F), jnp.float32),
    ),
    scratch_types=[
        pltpu.VMEM_SHARED((N_PAD, F), jnp.float32),
        pltpu.VMEM((8, CH), jnp.int32),
        pltpu.VMEM((8, CH), jnp.int32),
        pltpu.VMEM((CH, F), jnp.float32),
        pltpu.VMEM((CH, F), jnp.float32),
        pltpu.SemaphoreType.DMA,
        pltpu.SemaphoreType.DMA,
    ],
)
def _sc_layer1(masked_hbm, mask_hbm, src2d_hbm, dst2d_hbm, zeros128_hbm,
               agg_out, denom_out,
               acc, sidx, didx, rows_a, rows_b, sem_a, sem_b):
    c = lax.axis_index("c")
    s = lax.axis_index("s")
    nb = s * ROWS_PER_TILE

    pltpu.sync_copy(zeros128_hbm.at[pl.ds(nb, ROWS_PER_TILE)],
                    acc.at[pl.ds(nb, ROWS_PER_TILE)])

    plsc.subcore_barrier()

    def edge_loop(table_hbm):
        def body(j, carry):
            base = s * CHUNKS_B + j * 8
            pltpu.sync_copy(src2d_hbm.at[pl.ds(base, 8)], sidx)
            pltpu.sync_copy(dst2d_hbm.at[pl.ds(base, 8)], didx)
            bufs = ((rows_a, sem_a), (rows_b, sem_b))
            cp = [None] * 8
            cp[0] = pltpu.async_copy(table_hbm.at[sidx.at[0]], rows_a, sem_a)
            for k in range(8):
                if k + 1 < 8:
                    r, sm = bufs[(k + 1) % 2]
                    cp[k + 1] = pltpu.async_copy(
                        table_hbm.at[sidx.at[k + 1]], r, sm)
                r, _ = bufs[k % 2]
                cp[k].wait()
                pltpu.sync_copy(r, acc.at[didx.at[k]], add=True)
            return carry
        lax.fori_loop(0, CHUNKS_B // 8, body, 0)

    @pl.when(c == 0)
    def _():
        edge_loop(masked_hbm)

    @pl.when(c == 1)
    def _():
        edge_loop(mask_hbm)

    plsc.subcore_barrier()

    @pl.when(c == 0)
    def _():
        pltpu.sync_copy(acc.at[pl.ds(nb, ROWS_PER_TILE)],
                        agg_out.at[pl.ds(nb, ROWS_PER_TILE)])

    @pl.when(c == 1)
    def _():
        pltpu.sync_copy(acc.at[pl.ds(nb, ROWS_PER_TILE)],
                        denom_out.at[pl.ds(nb, ROWS_PER_TILE)])


@functools.partial(
    pl.kernel,
    mesh=_MESH,
    out_type=(
        jax.ShapeDtypeStruct((N_PAD, F), jnp.float32),
        jax.ShapeDtypeStruct((N_PAD, F), jnp.float32),
    ),
    scratch_types=[
        pltpu.VMEM_SHARED((N_PAD, F), jnp.float32),
        pltpu.VMEM((8, CH), jnp.int32),
        pltpu.VMEM((CH, F), jnp.float32),
    ],
)
def _sc_deg(dst2d_hbm, zeros128_hbm, ones128_hbm,
            d0_out, d1_out,
            acc, didx, ones_v):
    c = lax.axis_index("c")
    s = lax.axis_index("s")
    nb = s * ROWS_PER_TILE

    pltpu.sync_copy(zeros128_hbm.at[pl.ds(nb, ROWS_PER_TILE)],
                    acc.at[pl.ds(nb, ROWS_PER_TILE)])
    pltpu.sync_copy(ones128_hbm, ones_v)

    plsc.subcore_barrier()

    wrow = (c * NS + s) * CHUNKS_D

    def body(j, carry):
        pltpu.sync_copy(dst2d_hbm.at[pl.ds(wrow + j * 8, 8)], didx)
        for k in range(8):
            pltpu.sync_copy(ones_v, acc.at[didx.at[k]], add=True)
        return carry
    lax.fori_loop(0, CHUNKS_D // 8, body, 0)

    plsc.subcore_barrier()

    @pl.when(c == 0)
    def _():
        pltpu.sync_copy(acc.at[pl.ds(nb, ROWS_PER_TILE)],
                        d0_out.at[pl.ds(nb, ROWS_PER_TILE)])

    @pl.when(c == 1)
    def _():
        pltpu.sync_copy(acc.at[pl.ds(nb, ROWS_PER_TILE)],
                        d1_out.at[pl.ds(nb, ROWS_PER_TILE)])


@functools.partial(
    pl.kernel,
    mesh=_MESH,
    out_type=(
        jax.ShapeDtypeStruct((N_PAD, F), jnp.float32),
        jax.ShapeDtypeStruct((N_PAD, F), jnp.float32),
    ),
    scratch_types=[
        pltpu.VMEM_SHARED((N_PAD, F), jnp.float32),
        pltpu.VMEM((8, CH), jnp.int32),
        pltpu.VMEM((8, CH), jnp.int32),
        pltpu.VMEM((CH, F), jnp.float32),
        pltpu.VMEM((CH, F), jnp.float32),
        pltpu.SemaphoreType.DMA,
        pltpu.SemaphoreType.DMA,
    ],
)
def _sc_layer2(g_hbm, src2d_hbm, dst2d_hbm, zeros128_hbm,
               p0_out, p1_out,
               acc, sidx, didx, rows_a, rows_b, sem_a, sem_b):
    c = lax.axis_index("c")
    s = lax.axis_index("s")
    nb = s * ROWS_PER_TILE

    pltpu.sync_copy(zeros128_hbm.at[pl.ds(nb, ROWS_PER_TILE)],
                    acc.at[pl.ds(nb, ROWS_PER_TILE)])

    plsc.subcore_barrier()

    wrow = (c * NS + s) * CHUNKS_D

    def body(j, carry):
        base = wrow + j * 8
        pltpu.sync_copy(src2d_hbm.at[pl.ds(base, 8)], sidx)
        pltpu.sync_copy(dst2d_hbm.at[pl.ds(base, 8)], didx)
        bufs = ((rows_a, sem_a), (rows_b, sem_b))
        cp = [None] * 8
        cp[0] = pltpu.async_copy(g_hbm.at[sidx.at[0]], rows_a, sem_a)
        for k in range(8):
            if k + 1 < 8:
                r, sm = bufs[(k + 1) % 2]
                cp[k + 1] = pltpu.async_copy(g_hbm.at[sidx.at[k + 1]], r, sm)
            r, _ = bufs[k % 2]
            cp[k].wait()
            pltpu.sync_copy(r, acc.at[didx.at[k]], add=True)
        return carry
    lax.fori_loop(0, CHUNKS_D // 8, body, 0)

    plsc.subcore_barrier()

    @pl.when(c == 0)
    def _():
        pltpu.sync_copy(acc.at[pl.ds(nb, ROWS_PER_TILE)],
                        p0_out.at[pl.ds(nb, ROWS_PER_TILE)])

    @pl.when(c == 1)
    def _():
        pltpu.sync_copy(acc.at[pl.ds(nb, ROWS_PER_TILE)],
                        p1_out.at[pl.ds(nb, ROWS_PER_TILE)])


def _tc_masked_body(x_ref, m_ref, o_ref):
    o_ref[...] = x_ref[...] * m_ref[...]


def _tc_dense_body(agg_ref, den_ref, d0_ref, d1_ref, w1_ref, b1_ref, w2_ref,
                   g_ref):
    a = agg_ref[...] / jnp.maximum(den_ref[...], 1.0)
    h = jnp.dot(a, w1_ref[...], preferred_element_type=jnp.float32) + b1_ref[...]
    h = jnp.maximum(h, 0.0)
    hw = jnp.dot(h, w2_ref[...], preferred_element_type=jnp.float32)
    deg = d0_ref[...][:, 0:1] + d1_ref[...][:, 0:1]
    dinv = lax.rsqrt(deg + 1.0)
    gv = dinv * hw
    g_ref[...] = jnp.concatenate([gv, jnp.zeros_like(gv)], axis=1)


def _tc_final_body(p0_ref, p1_ref, g_ref, d0_ref, d1_ref, b2_ref, o_ref):
    deg = d0_ref[...][:, 0:1] + d1_ref[...][:, 0:1]
    dinv = lax.rsqrt(deg + 1.0)
    tot = (p0_ref[...] + p1_ref[...] + g_ref[...])[:, :C]
    o = dinv * tot + b2_ref[...]
    m = jnp.max(o, axis=1, keepdims=True)
    z = o - m
    lse = jnp.log(jnp.sum(jnp.exp(z), axis=1, keepdims=True))
    o_ref[...] = z - lse


def kernel(x, edge_index, mask, W1, b1, W2, b2):
    src, dst = edge_index[0], edge_index[1]

    pad_n = N_PAD - N
    xp = jnp.pad(x, ((0, pad_n), (0, 0)))
    maskp = jnp.pad(mask, ((0, pad_n), (0, 0)))
    pad_e = E_PAD - E
    fill = N + (jnp.arange(pad_e, dtype=jnp.int32) % (N_PAD - N))
    srcp = jnp.concatenate([src, fill]).reshape(-1, CH)
    dstp = jnp.concatenate([dst, fill]).reshape(-1, CH)
    zeros128 = jnp.zeros((N_PAD, F), jnp.float32)
    ones128 = jnp.ones((CH, F), jnp.float32)
    b1r = b1.reshape(1, HID)
    b2r = b2.reshape(1, C)

    grid = N_PAD // 1024
    masked = pl.pallas_call(
        _tc_masked_body,
        grid=(grid,),
        in_specs=[
            pl.BlockSpec((1024, F), lambda i: (i, 0)),
            pl.BlockSpec((1024, F), lambda i: (i, 0)),
        ],
        out_specs=pl.BlockSpec((1024, F), lambda i: (i, 0)),
        out_shape=jax.ShapeDtypeStruct((N_PAD, F), jnp.float32),
    )(xp, maskp)

    d0, d1 = _sc_deg(dstp, zeros128, ones128)
    agg, denom = _sc_layer1(masked, maskp, srcp, dstp, zeros128)

    g = pl.pallas_call(
        _tc_dense_body,
        grid=(grid,),
        in_specs=[
            pl.BlockSpec((1024, F), lambda i: (i, 0)),
            pl.BlockSpec((1024, F), lambda i: (i, 0)),
            pl.BlockSpec((1024, F), lambda i: (i, 0)),
            pl.BlockSpec((1024, F), lambda i: (i, 0)),
            pl.BlockSpec((F, HID), lambda i: (0, 0)),
            pl.BlockSpec((1, HID), lambda i: (0, 0)),
            pl.BlockSpec((HID, C), lambda i: (0, 0)),
        ],
        out_specs=pl.BlockSpec((1024, F), lambda i: (i, 0)),
        out_shape=jax.ShapeDtypeStruct((N_PAD, F), jnp.float32),
    )(agg, denom, d0, d1, W1, b1r, W2)

    p0, p1 = _sc_layer2(g, srcp, dstp, zeros128)

    out = pl.pallas_call(
        _tc_final_body,
        grid=(10,),
        in_specs=[
            pl.BlockSpec((1000, F), lambda i: (i, 0)),
            pl.BlockSpec((1000, F), lambda i: (i, 0)),
            pl.BlockSpec((1000, F), lambda i: (i, 0)),
            pl.BlockSpec((1000, F), lambda i: (i, 0)),
            pl.BlockSpec((1000, F), lambda i: (i, 0)),
            pl.BlockSpec((1, C), lambda i: (0, 0)),
        ],
        out_specs=pl.BlockSpec((1000, C), lambda i: (i, 0)),
        out_shape=jax.ShapeDtypeStruct((N, C), jnp.float32),
    )(p0, p1, g, d0, d1, b2r)

    return out


# speedup vs baseline: 18.5958x; 1.1575x over previous
import functools

import jax
import jax.numpy as jnp
from jax import lax
from jax.experimental import pallas as pl
from jax.experimental.pallas import tpu as pltpu
from jax.experimental.pallas import tpu_sc as plsc

N = 10000
E = 320000
F = 128
HID = 128
C = 64

NC = 2
NS = 16
CH = 128

N_PAD = 10240
E_PAD = 327680
ROWS_PER_TILE = N_PAD // NS
CHUNKS_B = E_PAD // (NS * CH)
CHUNKS_D = E_PAD // (NC * NS * CH)

_MESH = plsc.VectorSubcoreMesh(
    core_axis_name="c", subcore_axis_name="s", num_cores=NC, num_subcores=NS
)


@functools.partial(
    pl.kernel,
    mesh=_MESH,
    compiler_params=pltpu.CompilerParams(needs_layout_passes=False),
    out_type=(
        jax.ShapeDtypeStruct((N_PAD, F), jnp.float32),
        jax.ShapeDtypeStruct((N_PAD, F), jnp.float32),
        jax.ShapeDtypeStruct((N_PAD // CH, CH), jnp.float32),
    ),
    scratch_types=[
        pltpu.VMEM_SHARED((N_PAD, F), jnp.float32),
        pltpu.VMEM_SHARED((N_PAD // CH, CH), jnp.float32),
        pltpu.VMEM((8, CH), jnp.int32),
        pltpu.VMEM((8, CH), jnp.int32),
        pltpu.VMEM((CH, F), jnp.float32),
        pltpu.VMEM((CH, F), jnp.float32),
        pltpu.VMEM((N_PAD // CH, CH), jnp.float32),
        pltpu.VMEM((1, N_PAD // CH), jnp.int32),
        pltpu.SemaphoreType.DMA,
        pltpu.SemaphoreType.DMA,
    ],
)
def _sc_layer1(masked_hbm, mask_hbm, src2d_hbm, dst2d_hbm, zeros_hbm,
               agg_out, denom_out, deg_out,
               acc, degacc, sidx, didx, rows_a, rows_b, hist, rowidx,
               sem_a, sem_b):
    c = lax.axis_index("c")
    s = lax.axis_index("s")
    nb = s * ROWS_PER_TILE
    nrows = N_PAD // CH

    pltpu.sync_copy(zeros_hbm, acc.at[pl.ds(nb, ROWS_PER_TILE)])

    @pl.when(c == 1)
    def _():
        pltpu.sync_copy(zeros_hbm.at[pl.ds(0, nrows)], hist)
        for l in range(nrows // 16):
            rowidx[0, pl.ds(l * 16, 16)] = (
                lax.broadcasted_iota(jnp.int32, (16,), 0) + l * 16)

        @pl.when(s == 0)
        def _():
            pltpu.sync_copy(zeros_hbm.at[pl.ds(0, nrows)], degacc)

    plsc.subcore_barrier()

    def edge_loop(table_hbm, with_hist):
        def body(j, carry):
            base = s * CHUNKS_B + j * 8
            pltpu.sync_copy(src2d_hbm.at[pl.ds(base, 8)], sidx)
            pltpu.sync_copy(dst2d_hbm.at[pl.ds(base, 8)], didx)
            bufs = ((rows_a, sem_a), (rows_b, sem_b))
            cp = [None] * 8
            cp[0] = pltpu.async_copy(table_hbm.at[sidx.at[0]], rows_a, sem_a)
            for k in range(8):
                if k + 1 < 8:
                    r, sm = bufs[(k + 1) % 2]
                    cp[k + 1] = pltpu.async_copy(
                        table_hbm.at[sidx.at[k + 1]], r, sm)
                r, _ = bufs[k % 2]
                cp[k].wait()
                pltpu.sync_copy(r, acc.at[didx.at[k]], add=True)
                if with_hist:
                    for l in range(CH // 16):
                        idx16 = didx[k, pl.ds(l * 16, 16)]
                        cnt, last = plsc.scan_count(idx16)
                        plsc.addupdate_scatter(
                            hist,
                            [lax.shift_right_logical(idx16, 7),
                             lax.bitwise_and(idx16, 127)],
                            cnt.astype(jnp.float32), mask=last)
            return carry
        lax.fori_loop(0, CHUNKS_B // 8, body, 0)

    @pl.when(c == 0)
    def _():
        edge_loop(masked_hbm, False)

    @pl.when(c == 1)
    def _():
        edge_loop(mask_hbm, True)
        pltpu.sync_copy(hist, degacc.at[rowidx.at[0]], add=True)

    plsc.subcore_barrier()

    @pl.when(c == 0)
    def _():
        pltpu.sync_copy(acc.at[pl.ds(nb, ROWS_PER_TILE)],
                        agg_out.at[pl.ds(nb, ROWS_PER_TILE)])

    @pl.when(c == 1)
    def _():
        pltpu.sync_copy(acc.at[pl.ds(nb, ROWS_PER_TILE)],
                        denom_out.at[pl.ds(nb, ROWS_PER_TILE)])

        @pl.when(s == 0)
        def _():
            pltpu.sync_copy(degacc, deg_out)


@functools.partial(
    pl.kernel,
    mesh=_MESH,
    out_type=(
        jax.ShapeDtypeStruct((N_PAD, F), jnp.float32),
        jax.ShapeDtypeStruct((N_PAD, F), jnp.float32),
    ),
    scratch_types=[
        pltpu.VMEM_SHARED((N_PAD, F), jnp.float32),
        pltpu.VMEM((8, CH), jnp.int32),
        pltpu.VMEM((8, CH), jnp.int32),
        pltpu.VMEM((CH, F), jnp.float32),
        pltpu.VMEM((CH, F), jnp.float32),
        pltpu.SemaphoreType.DMA,
        pltpu.SemaphoreType.DMA,
    ],
)
def _sc_layer2(g_hbm, src2d_hbm, dst2d_hbm, zeros_hbm,
               p0_out, p1_out,
               acc, sidx, didx, rows_a, rows_b, sem_a, sem_b):
    c = lax.axis_index("c")
    s = lax.axis_index("s")
    nb = s * ROWS_PER_TILE

    pltpu.sync_copy(zeros_hbm, acc.at[pl.ds(nb, ROWS_PER_TILE)])

    plsc.subcore_barrier()

    wrow = (c * NS + s) * CHUNKS_D

    def body(j, carry):
        base = wrow + j * 8
        pltpu.sync_copy(src2d_hbm.at[pl.ds(base, 8)], sidx)
        pltpu.sync_copy(dst2d_hbm.at[pl.ds(base, 8)], didx)
        bufs = ((rows_a, sem_a), (rows_b, sem_b))
        cp = [None] * 8
        cp[0] = pltpu.async_copy(g_hbm.at[sidx.at[0]], rows_a, sem_a)
        for k in range(8):
            if k + 1 < 8:
                r, sm = bufs[(k + 1) % 2]
                cp[k + 1] = pltpu.async_copy(g_hbm.at[sidx.at[k + 1]], r, sm)
            r, _ = bufs[k % 2]
            cp[k].wait()
            pltpu.sync_copy(r, acc.at[didx.at[k]], add=True)
        return carry
    lax.fori_loop(0, CHUNKS_D // 8, body, 0)

    plsc.subcore_barrier()

    @pl.when(c == 0)
    def _():
        pltpu.sync_copy(acc.at[pl.ds(nb, ROWS_PER_TILE)],
                        p0_out.at[pl.ds(nb, ROWS_PER_TILE)])

    @pl.when(c == 1)
    def _():
        pltpu.sync_copy(acc.at[pl.ds(nb, ROWS_PER_TILE)],
                        p1_out.at[pl.ds(nb, ROWS_PER_TILE)])


def _tc_masked_body(x_ref, m_ref, o_ref):
    o_ref[...] = x_ref[...] * m_ref[...]


def _tc_dense_body(agg_ref, den_ref, deg_ref, w1_ref, b1_ref, w2_ref,
                   g_ref):
    a = agg_ref[...] / jnp.maximum(den_ref[...], 1.0)
    h = jnp.dot(a, w1_ref[...], preferred_element_type=jnp.float32) + b1_ref[...]
    h = jnp.maximum(h, 0.0)
    hw = jnp.dot(h, w2_ref[...], preferred_element_type=jnp.float32)
    dinv = lax.rsqrt(deg_ref[...][:, 0:1] + 1.0)
    gv = dinv * hw
    g_ref[...] = jnp.concatenate([gv, jnp.zeros_like(gv)], axis=1)


def _tc_final_body(p0_ref, p1_ref, g_ref, deg_ref, b2_ref, o_ref):
    dinv = lax.rsqrt(deg_ref[...][:, 0:1] + 1.0)
    tot = (p0_ref[...] + p1_ref[...] + g_ref[...])[:, :C]
    o = dinv * tot + b2_ref[...]
    m = jnp.max(o, axis=1, keepdims=True)
    z = o - m
    lse = jnp.log(jnp.sum(jnp.exp(z), axis=1, keepdims=True))
    o_ref[...] = z - lse


def kernel(x, edge_index, mask, W1, b1, W2, b2):
    src, dst = edge_index[0], edge_index[1]

    pad_e = E_PAD - E
    ar = jnp.arange(pad_e, dtype=jnp.int32)
    srcp = jnp.concatenate([src, ar % 2048]).reshape(-1, CH)
    dstp = jnp.concatenate([dst, N + (ar % (N_PAD - N))]).reshape(-1, CH)
    zeros640 = jnp.zeros((ROWS_PER_TILE, F), jnp.float32)
    b1r = b1.reshape(1, HID)
    b2r = b2.reshape(1, C)

    masked = pl.pallas_call(
        _tc_masked_body,
        grid=(10,),
        in_specs=[
            pl.BlockSpec((1000, F), lambda i: (i, 0)),
            pl.BlockSpec((1000, F), lambda i: (i, 0)),
        ],
        out_specs=pl.BlockSpec((1000, F), lambda i: (i, 0)),
        out_shape=jax.ShapeDtypeStruct((N, F), jnp.float32),
    )(x, mask)

    agg, denom, degp = _sc_layer1(masked, mask, srcp, dstp, zeros640)
    deg8 = jnp.broadcast_to(degp.reshape(N_PAD)[:, None], (N_PAD, 8))
    grid = N_PAD // 1024

    g = pl.pallas_call(
        _tc_dense_body,
        grid=(grid,),
        in_specs=[
            pl.BlockSpec((1024, F), lambda i: (i, 0)),
            pl.BlockSpec((1024, F), lambda i: (i, 0)),
            pl.BlockSpec((1024, 8), lambda i: (i, 0)),
            pl.BlockSpec((F, HID), lambda i: (0, 0)),
            pl.BlockSpec((1, HID), lambda i: (0, 0)),
            pl.BlockSpec((HID, C), lambda i: (0, 0)),
        ],
        out_specs=pl.BlockSpec((1024, F), lambda i: (i, 0)),
        out_shape=jax.ShapeDtypeStruct((N_PAD, F), jnp.float32),
    )(agg, denom, deg8, W1, b1r, W2)

    p0, p1 = _sc_layer2(g, srcp, dstp, zeros640)

    out = pl.pallas_call(
        _tc_final_body,
        grid=(10,),
        in_specs=[
            pl.BlockSpec((1000, F), lambda i: (i, 0)),
            pl.BlockSpec((1000, F), lambda i: (i, 0)),
            pl.BlockSpec((1000, F), lambda i: (i, 0)),
            pl.BlockSpec((1000, 8), lambda i: (i, 0)),
            pl.BlockSpec((1, C), lambda i: (0, 0)),
        ],
        out_specs=pl.BlockSpec((1000, C), lambda i: (i, 0)),
        out_shape=jax.ShapeDtypeStruct((N, C), jnp.float32),
    )(p0, p1, g, deg8, b2r)

    return out


# speedup vs baseline: 18.8474x; 1.0135x over previous
import functools

import jax
import jax.numpy as jnp
from jax import lax
from jax.experimental import pallas as pl
from jax.experimental.pallas import tpu as pltpu
from jax.experimental.pallas import tpu_sc as plsc

N = 10000
E = 320000
F = 128
HID = 128
C = 64

NC = 2
NS = 16
CH = 128

N_PAD = 10240
E_PAD = 327680
ROWS_PER_TILE = N_PAD // NS
CHUNKS_B = E_PAD // (NS * CH)
CHUNKS_D = E_PAD // (NC * NS * CH)

_MESH = plsc.VectorSubcoreMesh(
    core_axis_name="c", subcore_axis_name="s", num_cores=NC, num_subcores=NS
)


@functools.partial(
    pl.kernel,
    mesh=_MESH,
    compiler_params=pltpu.CompilerParams(needs_layout_passes=False),
    out_type=(
        jax.ShapeDtypeStruct((N_PAD, F), jnp.float32),
        jax.ShapeDtypeStruct((N_PAD, F), jnp.float32),
        jax.ShapeDtypeStruct((N_PAD // CH, CH), jnp.float32),
    ),
    scratch_types=[
        pltpu.VMEM_SHARED((N_PAD, F), jnp.float32),
        pltpu.VMEM_SHARED((N_PAD // CH, CH), jnp.float32),
        pltpu.VMEM((8, CH), jnp.int32),
        pltpu.VMEM((8, CH), jnp.int32),
        pltpu.VMEM((CH, F), jnp.float32),
        pltpu.VMEM((CH, F), jnp.float32),
        pltpu.VMEM((N_PAD // CH, CH), jnp.float32),
        pltpu.VMEM((1, N_PAD // CH), jnp.int32),
        pltpu.SemaphoreType.DMA,
        pltpu.SemaphoreType.DMA,
        pltpu.SemaphoreType.DMA,
        pltpu.SemaphoreType.DMA,
    ],
)
def _sc_layer1(masked_hbm, mask_hbm, src2d_hbm, dst2d_hbm, zeros_hbm,
               agg_out, denom_out, deg_out,
               acc, degacc, sidx, didx, rows_a, rows_b, hist, rowidx,
               sem_a, sem_b, sem_sa, sem_sb):
    c = lax.axis_index("c")
    s = lax.axis_index("s")
    nb = s * ROWS_PER_TILE
    nrows = N_PAD // CH

    pltpu.sync_copy(zeros_hbm, acc.at[pl.ds(nb, ROWS_PER_TILE)])

    @pl.when(c == 1)
    def _():
        pltpu.sync_copy(zeros_hbm.at[pl.ds(0, nrows)], hist)
        for l in range(nrows // 16):
            rowidx[0, pl.ds(l * 16, 16)] = (
                lax.broadcasted_iota(jnp.int32, (16,), 0) + l * 16)

        @pl.when(s == 0)
        def _():
            pltpu.sync_copy(zeros_hbm.at[pl.ds(0, nrows)], degacc)

    plsc.subcore_barrier()

    def edge_loop(table_hbm, with_hist):
        def body(j, carry):
            base = s * CHUNKS_B + j * 8
            pltpu.sync_copy(src2d_hbm.at[pl.ds(base, 8)], sidx)
            pltpu.sync_copy(dst2d_hbm.at[pl.ds(base, 8)], didx)
            bufs = ((rows_a, sem_a, sem_sa), (rows_b, sem_b, sem_sb))
            cp = [None] * 8
            sc = [None] * 8
            cp[0] = pltpu.async_copy(table_hbm.at[sidx.at[0]], rows_a, sem_a)
            cp[1] = pltpu.async_copy(table_hbm.at[sidx.at[1]], rows_b, sem_b)
            for k in range(8):
                r, _, sms = bufs[k % 2]
                cp[k].wait()
                sc[k] = pltpu.async_copy(r, acc.at[didx.at[k]], sms,
                                         add=True)
                if k + 2 < 8:
                    sc[k].wait()
                    cp[k + 2] = pltpu.async_copy(
                        table_hbm.at[sidx.at[k + 2]], r, bufs[k % 2][1])
                if with_hist:
                    for l in range(CH // 16):
                        idx16 = didx[k, pl.ds(l * 16, 16)]
                        cnt, last = plsc.scan_count(idx16)
                        plsc.addupdate_scatter(
                            hist,
                            [lax.shift_right_logical(idx16, 7),
                             lax.bitwise_and(idx16, 127)],
                            cnt.astype(jnp.float32), mask=last)
            sc[6].wait()
            sc[7].wait()
            return carry
        lax.fori_loop(0, CHUNKS_B // 8, body, 0)

    @pl.when(c == 0)
    def _():
        edge_loop(masked_hbm, False)

    @pl.when(c == 1)
    def _():
        edge_loop(mask_hbm, True)
        pltpu.sync_copy(hist, degacc.at[rowidx.at[0]], add=True)

    plsc.subcore_barrier()

    @pl.when(c == 0)
    def _():
        pltpu.sync_copy(acc.at[pl.ds(nb, ROWS_PER_TILE)],
                        agg_out.at[pl.ds(nb, ROWS_PER_TILE)])

    @pl.when(c == 1)
    def _():
        pltpu.sync_copy(acc.at[pl.ds(nb, ROWS_PER_TILE)],
                        denom_out.at[pl.ds(nb, ROWS_PER_TILE)])

        @pl.when(s == 0)
        def _():
            pltpu.sync_copy(degacc, deg_out)


@functools.partial(
    pl.kernel,
    mesh=_MESH,
    out_type=(
        jax.ShapeDtypeStruct((N_PAD, F), jnp.float32),
        jax.ShapeDtypeStruct((N_PAD, F), jnp.float32),
    ),
    scratch_types=[
        pltpu.VMEM_SHARED((N_PAD, F), jnp.float32),
        pltpu.VMEM((8, CH), jnp.int32),
        pltpu.VMEM((8, CH), jnp.int32),
        pltpu.VMEM((CH, F), jnp.float32),
        pltpu.VMEM((CH, F), jnp.float32),
        pltpu.SemaphoreType.DMA,
        pltpu.SemaphoreType.DMA,
        pltpu.SemaphoreType.DMA,
        pltpu.SemaphoreType.DMA,
    ],
)
def _sc_layer2(g_hbm, src2d_hbm, dst2d_hbm, zeros_hbm,
               p0_out, p1_out,
               acc, sidx, didx, rows_a, rows_b, sem_a, sem_b, sem_sa, sem_sb):
    c = lax.axis_index("c")
    s = lax.axis_index("s")
    nb = s * ROWS_PER_TILE

    pltpu.sync_copy(zeros_hbm, acc.at[pl.ds(nb, ROWS_PER_TILE)])

    plsc.subcore_barrier()

    wrow = (c * NS + s) * CHUNKS_D

    def body(j, carry):
        base = wrow + j * 8
        pltpu.sync_copy(src2d_hbm.at[pl.ds(base, 8)], sidx)
        pltpu.sync_copy(dst2d_hbm.at[pl.ds(base, 8)], didx)
        bufs = ((rows_a, sem_a, sem_sa), (rows_b, sem_b, sem_sb))
        cp = [None] * 8
        sc = [None] * 8
        cp[0] = pltpu.async_copy(g_hbm.at[sidx.at[0]], rows_a, sem_a)
        cp[1] = pltpu.async_copy(g_hbm.at[sidx.at[1]], rows_b, sem_b)
        for k in range(8):
            r, _, sms = bufs[k % 2]
            cp[k].wait()
            sc[k] = pltpu.async_copy(r, acc.at[didx.at[k]], sms, add=True)
            if k + 2 < 8:
                sc[k].wait()
                cp[k + 2] = pltpu.async_copy(
                    g_hbm.at[sidx.at[k + 2]], r, bufs[k % 2][1])
        sc[6].wait()
        sc[7].wait()
        return carry
    lax.fori_loop(0, CHUNKS_D // 8, body, 0)

    plsc.subcore_barrier()

    @pl.when(c == 0)
    def _():
        pltpu.sync_copy(acc.at[pl.ds(nb, ROWS_PER_TILE)],
                        p0_out.at[pl.ds(nb, ROWS_PER_TILE)])

    @pl.when(c == 1)
    def _():
        pltpu.sync_copy(acc.at[pl.ds(nb, ROWS_PER_TILE)],
                        p1_out.at[pl.ds(nb, ROWS_PER_TILE)])


def _tc_masked_body(x_ref, m_ref, o_ref):
    o_ref[...] = x_ref[...] * m_ref[...]


def _tc_dense_body(agg_ref, den_ref, deg_ref, w1_ref, b1_ref, w2_ref,
                   g_ref):
    a = agg_ref[...] / jnp.maximum(den_ref[...], 1.0)
    h = jnp.dot(a, w1_ref[...], preferred_element_type=jnp.float32) + b1_ref[...]
    h = jnp.maximum(h, 0.0)
    hw = jnp.dot(h, w2_ref[...], preferred_element_type=jnp.float32)
    dinv = lax.rsqrt(deg_ref[...][:, 0:1] + 1.0)
    gv = dinv * hw
    g_ref[...] = jnp.concatenate([gv, jnp.zeros_like(gv)], axis=1)


def _tc_final_body(p0_ref, p1_ref, g_ref, deg_ref, b2_ref, o_ref):
    dinv = lax.rsqrt(deg_ref[...][:, 0:1] + 1.0)
    tot = (p0_ref[...] + p1_ref[...] + g_ref[...])[:, :C]
    o = dinv * tot + b2_ref[...]
    m = jnp.max(o, axis=1, keepdims=True)
    z = o - m
    lse = jnp.log(jnp.sum(jnp.exp(z), axis=1, keepdims=True))
    o_ref[...] = z - lse


def kernel(x, edge_index, mask, W1, b1, W2, b2):
    src, dst = edge_index[0], edge_index[1]

    pad_e = E_PAD - E
    ar = jnp.arange(pad_e, dtype=jnp.int32)
    srcp = jnp.concatenate([src, ar % 2048]).reshape(-1, CH)
    dstp = jnp.concatenate([dst, N + (ar % (N_PAD - N))]).reshape(-1, CH)
    zeros640 = jnp.zeros((ROWS_PER_TILE, F), jnp.float32)
    b1r = b1.reshape(1, HID)
    b2r = b2.reshape(1, C)

    masked = pl.pallas_call(
        _tc_masked_body,
        grid=(10,),
        in_specs=[
            pl.BlockSpec((1000, F), lambda i: (i, 0)),
            pl.BlockSpec((1000, F), lambda i: (i, 0)),
        ],
        out_specs=pl.BlockSpec((1000, F), lambda i: (i, 0)),
        out_shape=jax.ShapeDtypeStruct((N, F), jnp.float32),
    )(x, mask)

    agg, denom, degp = _sc_layer1(masked, mask, srcp, dstp, zeros640)
    deg8 = jnp.broadcast_to(degp.reshape(N_PAD)[:, None], (N_PAD, 8))
    grid = N_PAD // 1024

    g = pl.pallas_call(
        _tc_dense_body,
        grid=(grid,),
        in_specs=[
            pl.BlockSpec((1024, F), lambda i: (i, 0)),
            pl.BlockSpec((1024, F), lambda i: (i, 0)),
            pl.BlockSpec((1024, 8), lambda i: (i, 0)),
            pl.BlockSpec((F, HID), lambda i: (0, 0)),
            pl.BlockSpec((1, HID), lambda i: (0, 0)),
            pl.BlockSpec((HID, C), lambda i: (0, 0)),
        ],
        out_specs=pl.BlockSpec((1024, F), lambda i: (i, 0)),
        out_shape=jax.ShapeDtypeStruct((N_PAD, F), jnp.float32),
    )(agg, denom, deg8, W1, b1r, W2)

    p0, p1 = _sc_layer2(g, srcp, dstp, zeros640)

    out = pl.pallas_call(
        _tc_final_body,
        grid=(10,),
        in_specs=[
            pl.BlockSpec((1000, F), lambda i: (i, 0)),
            pl.BlockSpec((1000, F), lambda i: (i, 0)),
            pl.BlockSpec((1000, F), lambda i: (i, 0)),
            pl.BlockSpec((1000, 8), lambda i: (i, 0)),
            pl.BlockSpec((1, C), lambda i: (0, 0)),
        ],
        out_specs=pl.BlockSpec((1000, C), lambda i: (i, 0)),
        out_shape=jax.ShapeDtypeStruct((N, C), jnp.float32),
    )(p0, p1, g, deg8, b2r)

    return out


# speedup vs baseline: 20.1049x; 1.0667x over previous
import functools

import jax
import jax.numpy as jnp
from jax import lax
from jax.experimental import pallas as pl
from jax.experimental.pallas import tpu as pltpu
from jax.experimental.pallas import tpu_sc as plsc

N = 10000
E = 320000
F = 128
HID = 128
C = 64

NC = 2
NS = 16
CH = 128

N_PAD = 10240
E_PAD = 327680
ROWS_PER_TILE = N_PAD // NS
CHUNKS_B = E_PAD // (NS * CH)
CHUNKS_D = E_PAD // (NC * NS * CH)

_MESH = plsc.VectorSubcoreMesh(
    core_axis_name="c", subcore_axis_name="s", num_cores=NC, num_subcores=NS
)


@functools.partial(
    pl.kernel,
    mesh=_MESH,
    compiler_params=pltpu.CompilerParams(needs_layout_passes=False),
    out_type=(
        jax.ShapeDtypeStruct((N_PAD, F), jnp.float32),
        jax.ShapeDtypeStruct((N_PAD, F), jnp.float32),
        jax.ShapeDtypeStruct((N_PAD // CH, CH), jnp.float32),
    ),
    scratch_types=[
        pltpu.VMEM_SHARED((N_PAD, F), jnp.float32),
        pltpu.VMEM_SHARED((N_PAD // CH, CH), jnp.float32),
        pltpu.VMEM((8, 2, CH), jnp.int32),
        pltpu.VMEM((8, 2, CH), jnp.int32),
        pltpu.VMEM((CH, F), jnp.float32),
        pltpu.VMEM((CH, F), jnp.float32),
        pltpu.VMEM((N_PAD // CH, CH), jnp.float32),
        pltpu.VMEM((1, N_PAD // CH), jnp.int32),
        pltpu.SemaphoreType.DMA,
        pltpu.SemaphoreType.DMA,
        pltpu.SemaphoreType.DMA,
        pltpu.SemaphoreType.DMA,
        pltpu.SemaphoreType.DMA,
        pltpu.SemaphoreType.DMA,
    ],
)
def _sc_layer1(masked_hbm, mask_hbm, edges_hbm, zeros_hbm,
               agg_out, denom_out, deg_out,
               acc, degacc, eidx_a, eidx_b, rows_a, rows_b, hist, rowidx,
               sem_a, sem_b, sem_sa, sem_sb, sem_ia, sem_ib):
    c = lax.axis_index("c")
    s = lax.axis_index("s")
    nb = s * ROWS_PER_TILE
    nrows = N_PAD // CH

    pltpu.sync_copy(zeros_hbm, acc.at[pl.ds(nb, ROWS_PER_TILE)])

    @pl.when(c == 1)
    def _():
        pltpu.sync_copy(zeros_hbm.at[pl.ds(0, nrows)], hist)
        for l in range(nrows // 16):
            rowidx[0, pl.ds(l * 16, 16)] = (
                lax.broadcasted_iota(jnp.int32, (16,), 0) + l * 16)

        @pl.when(s == 0)
        def _():
            pltpu.sync_copy(zeros_hbm.at[pl.ds(0, nrows)], degacc)

    plsc.subcore_barrier()

    def edge_loop(table_hbm, with_hist):
        base0 = s * CHUNKS_B

        def process(eidx):
            bufs = ((rows_a, sem_a, sem_sa), (rows_b, sem_b, sem_sb))
            cp = [None] * 8
            sc = [None] * 8
            cp[0] = pltpu.async_copy(table_hbm.at[eidx.at[0, 0]],
                                     rows_a, sem_a)
            cp[1] = pltpu.async_copy(table_hbm.at[eidx.at[1, 0]],
                                     rows_b, sem_b)
            for k in range(8):
                r, _, sms = bufs[k % 2]
                cp[k].wait()
                sc[k] = pltpu.async_copy(r, acc.at[eidx.at[k, 1]], sms,
                                         add=True)
                if k + 2 < 8:
                    sc[k].wait()
                    cp[k + 2] = pltpu.async_copy(
                        table_hbm.at[eidx.at[k + 2, 0]], r, bufs[k % 2][1])
                if with_hist:
                    for l in range(CH // 16):
                        idx16 = eidx[k, 1, pl.ds(l * 16, 16)]
                        cnt, last = plsc.scan_count(idx16)
                        plsc.addupdate_scatter(
                            hist,
                            [lax.shift_right_logical(idx16, 7),
                             lax.bitwise_and(idx16, 127)],
                            cnt.astype(jnp.float32), mask=last)
            sc[6].wait()
            sc[7].wait()

        pltpu.async_copy(edges_hbm.at[pl.ds(base0, 8)], eidx_a, sem_ia)
        ngroups = CHUNKS_B // 8

        def body(j2, carry):
            gb = base0 + j2 * 16
            pltpu.async_copy(edges_hbm.at[pl.ds(gb + 8, 8)], eidx_b, sem_ib)
            pltpu.make_async_copy(edges_hbm.at[pl.ds(gb, 8)],
                                  eidx_a, sem_ia).wait()
            process(eidx_a)

            @pl.when(j2 < ngroups // 2 - 1)
            def _():
                pltpu.async_copy(edges_hbm.at[pl.ds(gb + 16, 8)],
                                 eidx_a, sem_ia)
            pltpu.make_async_copy(edges_hbm.at[pl.ds(gb + 8, 8)],
                                  eidx_b, sem_ib).wait()
            process(eidx_b)
            return carry
        lax.fori_loop(0, ngroups // 2, body, 0)

    @pl.when(c == 0)
    def _():
        edge_loop(masked_hbm, False)

    @pl.when(c == 1)
    def _():
        edge_loop(mask_hbm, True)
        pltpu.sync_copy(hist, degacc.at[rowidx.at[0]], add=True)

    plsc.subcore_barrier()

    @pl.when(c == 0)
    def _():
        pltpu.sync_copy(acc.at[pl.ds(nb, ROWS_PER_TILE)],
                        agg_out.at[pl.ds(nb, ROWS_PER_TILE)])

    @pl.when(c == 1)
    def _():
        pltpu.sync_copy(acc.at[pl.ds(nb, ROWS_PER_TILE)],
                        denom_out.at[pl.ds(nb, ROWS_PER_TILE)])

        @pl.when(s == 0)
        def _():
            pltpu.sync_copy(degacc, deg_out)


@functools.partial(
    pl.kernel,
    mesh=_MESH,
    out_type=(
        jax.ShapeDtypeStruct((N_PAD, F), jnp.float32),
        jax.ShapeDtypeStruct((N_PAD, F), jnp.float32),
    ),
    scratch_types=[
        pltpu.VMEM_SHARED((N_PAD, F), jnp.float32),
        pltpu.VMEM((8, 2, CH), jnp.int32),
        pltpu.VMEM((8, 2, CH), jnp.int32),
        pltpu.VMEM((CH, F), jnp.float32),
        pltpu.VMEM((CH, F), jnp.float32),
        pltpu.SemaphoreType.DMA,
        pltpu.SemaphoreType.DMA,
        pltpu.SemaphoreType.DMA,
        pltpu.SemaphoreType.DMA,
        pltpu.SemaphoreType.DMA,
        pltpu.SemaphoreType.DMA,
    ],
)
def _sc_layer2(g_hbm, edges_hbm, zeros_hbm,
               p0_out, p1_out,
               acc, eidx_a, eidx_b, rows_a, rows_b,
               sem_a, sem_b, sem_sa, sem_sb, sem_ia, sem_ib):
    c = lax.axis_index("c")
    s = lax.axis_index("s")
    nb = s * ROWS_PER_TILE

    pltpu.sync_copy(zeros_hbm, acc.at[pl.ds(nb, ROWS_PER_TILE)])

    plsc.subcore_barrier()

    base0 = (c * NS + s) * CHUNKS_D

    def process(eidx):
        bufs = ((rows_a, sem_a, sem_sa), (rows_b, sem_b, sem_sb))
        cp = [None] * 8
        sc = [None] * 8
        cp[0] = pltpu.async_copy(g_hbm.at[eidx.at[0, 0]], rows_a, sem_a)
        cp[1] = pltpu.async_copy(g_hbm.at[eidx.at[1, 0]], rows_b, sem_b)
        for k in range(8):
            r, _, sms = bufs[k % 2]
            cp[k].wait()
            sc[k] = pltpu.async_copy(r, acc.at[eidx.at[k, 1]], sms, add=True)
            if k + 2 < 8:
                sc[k].wait()
                cp[k + 2] = pltpu.async_copy(
                    g_hbm.at[eidx.at[k + 2, 0]], r, bufs[k % 2][1])
        sc[6].wait()
        sc[7].wait()

    pltpu.async_copy(edges_hbm.at[pl.ds(base0, 8)], eidx_a, sem_ia)
    ngroups = CHUNKS_D // 8

    def body(j2, carry):
        gb = base0 + j2 * 16
        pltpu.async_copy(edges_hbm.at[pl.ds(gb + 8, 8)], eidx_b, sem_ib)
        pltpu.make_async_copy(edges_hbm.at[pl.ds(gb, 8)],
                              eidx_a, sem_ia).wait()
        process(eidx_a)

        @pl.when(j2 < ngroups // 2 - 1)
        def _():
            pltpu.async_copy(edges_hbm.at[pl.ds(gb + 16, 8)],
                             eidx_a, sem_ia)
        pltpu.make_async_copy(edges_hbm.at[pl.ds(gb + 8, 8)],
                              eidx_b, sem_ib).wait()
        process(eidx_b)
        return carry
    lax.fori_loop(0, ngroups // 2, body, 0)

    plsc.subcore_barrier()

    @pl.when(c == 0)
    def _():
        pltpu.sync_copy(acc.at[pl.ds(nb, ROWS_PER_TILE)],
                        p0_out.at[pl.ds(nb, ROWS_PER_TILE)])

    @pl.when(c == 1)
    def _():
        pltpu.sync_copy(acc.at[pl.ds(nb, ROWS_PER_TILE)],
                        p1_out.at[pl.ds(nb, ROWS_PER_TILE)])


def _tc_masked_body(x_ref, m_ref, o_ref):
    o_ref[...] = x_ref[...] * m_ref[...]


def _tc_dense_body(agg_ref, den_ref, deg_ref, w1_ref, b1_ref, w2_ref,
                   g_ref):
    a = agg_ref[...] / jnp.maximum(den_ref[...], 1.0)
    h = jnp.dot(a, w1_ref[...], preferred_element_type=jnp.float32) + b1_ref[...]
    h = jnp.maximum(h, 0.0)
    hw = jnp.dot(h, w2_ref[...], preferred_element_type=jnp.float32)
    dinv = lax.rsqrt(deg_ref[...][:, 0:1] + 1.0)
    gv = dinv * hw
    g_ref[...] = jnp.concatenate([gv, jnp.zeros_like(gv)], axis=1)


def _tc_final_body(p0_ref, p1_ref, g_ref, deg_ref, b2_ref, o_ref):
    dinv = lax.rsqrt(deg_ref[...][:, 0:1] + 1.0)
    tot = (p0_ref[...] + p1_ref[...] + g_ref[...])[:, :C]
    o = dinv * tot + b2_ref[...]
    m = jnp.max(o, axis=1, keepdims=True)
    z = o - m
    lse = jnp.log(jnp.sum(jnp.exp(z), axis=1, keepdims=True))
    o_ref[...] = z - lse


def kernel(x, edge_index, mask, W1, b1, W2, b2):
    src, dst = edge_index[0], edge_index[1]

    pad_e = E_PAD - E
    ar = jnp.arange(pad_e, dtype=jnp.int32)
    srcp = jnp.concatenate([src, ar % 2048]).reshape(-1, CH)
    dstp = jnp.concatenate([dst, N + (ar % (N_PAD - N))]).reshape(-1, CH)
    edges3d = jnp.stack([srcp, dstp], axis=1)
    zeros640 = jnp.zeros((ROWS_PER_TILE, F), jnp.float32)
    b1r = b1.reshape(1, HID)
    b2r = b2.reshape(1, C)

    masked = pl.pallas_call(
        _tc_masked_body,
        grid=(10,),
        in_specs=[
            pl.BlockSpec((1000, F), lambda i: (i, 0)),
            pl.BlockSpec((1000, F), lambda i: (i, 0)),
        ],
        out_specs=pl.BlockSpec((1000, F), lambda i: (i, 0)),
        out_shape=jax.ShapeDtypeStruct((N, F), jnp.float32),
    )(x, mask)

    agg, denom, degp = _sc_layer1(masked, mask, edges3d, zeros640)
    deg8 = jnp.broadcast_to(degp.reshape(N_PAD)[:, None], (N_PAD, 8))
    grid = N_PAD // 1024

    g = pl.pallas_call(
        _tc_dense_body,
        grid=(grid,),
        in_specs=[
            pl.BlockSpec((1024, F), lambda i: (i, 0)),
            pl.BlockSpec((1024, F), lambda i: (i, 0)),
            pl.BlockSpec((1024, 8), lambda i: (i, 0)),
            pl.BlockSpec((F, HID), lambda i: (0, 0)),
            pl.BlockSpec((1, HID), lambda i: (0, 0)),
            pl.BlockSpec((HID, C), lambda i: (0, 0)),
        ],
        out_specs=pl.BlockSpec((1024, F), lambda i: (i, 0)),
        out_shape=jax.ShapeDtypeStruct((N_PAD, F), jnp.float32),
    )(agg, denom, deg8, W1, b1r, W2)

    p0, p1 = _sc_layer2(g, edges3d, zeros640)

    out = pl.pallas_call(
        _tc_final_body,
        grid=(10,),
        in_specs=[
            pl.BlockSpec((1000, F), lambda i: (i, 0)),
            pl.BlockSpec((1000, F), lambda i: (i, 0)),
            pl.BlockSpec((1000, F), lambda i: (i, 0)),
            pl.BlockSpec((1000, 8), lambda i: (i, 0)),
            pl.BlockSpec((1, C), lambda i: (0, 0)),
        ],
        out_specs=pl.BlockSpec((1000, C), lambda i: (i, 0)),
        out_shape=jax.ShapeDtypeStruct((N, C), jnp.float32),
    )(p0, p1, g, deg8, b2r)

    return out
